# baseline (speedup 1.0000x reference)
import jax
import jax.numpy as jnp
from jax.experimental import pallas as pl
from jax.experimental.pallas import tpu as pltpu

_M_TILE = 4096
_N = 8192
_K = 32
_CHUNK = 2048
_NQ = _N // _CHUNK


def _prep_body(x_ref, w_ref, xs_ref, r_ref, wb_ref):
    x = x_ref[:]
    c = jnp.sqrt(jnp.sum(x * x, axis=0, keepdims=True))
    xs_ref[:] = (x / c).astype(jnp.bfloat16)
    w = w_ref[:]
    r_ref[:] = jnp.sqrt(jnp.sum(w * w, axis=1, keepdims=True))
    wb_ref[:] = w.astype(jnp.bfloat16)


def _argmax_body(xs_ref, wb_ref, r_ref, o_ref, accv_ref, acci_ref):
    q = pl.program_id(1)
    beta = jax.lax.dot_general(
        wb_ref[:], xs_ref[:], (((1,), (1,)), ((), ())),
        preferred_element_type=jnp.float32,
    )
    beta = beta / r_ref[:]
    m_q = jnp.max(beta, axis=0, keepdims=True)
    i_q = (jnp.argmax(beta, axis=0).astype(jnp.int32))[None, :] + q * _CHUNK

    @pl.when(q == 0)
    def _init():
        accv_ref[:] = m_q.astype(jnp.bfloat16).astype(jnp.float32)
        acci_ref[:] = i_q

    @pl.when(q > 0)
    def _fold():
        take = m_q > accv_ref[:]
        accv_ref[:] = jnp.where(
            take, m_q.astype(jnp.bfloat16).astype(jnp.float32), accv_ref[:])
        acci_ref[:] = jnp.where(take, i_q, acci_ref[:])

    @pl.when(q == _NQ - 1)
    def _out():
        o_ref[:] = acci_ref[:]


def kernel(all_ts, W, clustering_flag):
    del clustering_flag
    m = all_ts.shape[0]
    x = jnp.reshape(all_ts, (m, _K))

    xs16, r, wb = pl.pallas_call(
        _prep_body,
        out_shape=(
            jax.ShapeDtypeStruct((m, _K), jnp.bfloat16),
            jax.ShapeDtypeStruct((_N, 1), jnp.float32),
            jax.ShapeDtypeStruct((_N, _K), jnp.bfloat16),
        ),
    )(x, W)

    n_star = pl.pallas_call(
        _argmax_body,
        grid=(m // _M_TILE, _NQ),
        in_specs=[
            pl.BlockSpec((_M_TILE, _K), lambda i, q: (i, 0)),
            pl.BlockSpec((_CHUNK, _K), lambda i, q: (q, 0)),
            pl.BlockSpec((_CHUNK, 1), lambda i, q: (q, 0)),
        ],
        out_specs=pl.BlockSpec((1, _M_TILE), lambda i, q: (0, i)),
        out_shape=jax.ShapeDtypeStruct((1, m), jnp.int32),
        scratch_shapes=[
            pltpu.VMEM((1, _M_TILE), jnp.float32),
            pltpu.VMEM((1, _M_TILE), jnp.int32),
        ],
    )(xs16, wb, r)

    return jnp.reshape(n_star, (m,))
